# baseline (speedup 1.0000x reference)
import jax
import jax.numpy as jnp
import numpy as np
from jax.experimental import pallas as pl
from jax.experimental.pallas import tpu as pltpu
from jax.experimental.pallas import tpu_sc as plsc

_FMAX = np.float32(np.finfo(np.float32).max)
_FMAXBITS = np.int32(np.float32(_FMAX).view(np.int32))
_IMAX = np.int32(2**31 - 1)
_P = 1024
_W = 32
_L = 16


def _assign_body(pts_ref, anch_ref, aidx_ref, dbits_ref):
    P = pts_ref.shape[0]
    S = anch_ref.shape[1]

    pts = pts_ref[...]
    x = pts[:, 0:1]
    y = pts[:, 1:2]
    z = pts[:, 2:3]
    d = jnp.sqrt((x * x + y * y) + z * z)
    n = jnp.where(d == 0.0, jnp.float32(1.0), d)
    tp = (pts / n).astype(jnp.bfloat16)
    ab = anch_ref[...].astype(jnp.bfloat16)
    cos = jax.lax.dot_general(tp, ab, (((1,), (0,)), ((), ())),
                              preferred_element_type=jnp.float32)

    amax = jnp.max(cos, axis=1, keepdims=True)
    m = cos == amax

    srow = jax.lax.broadcasted_iota(jnp.int32, (S, 128), 0)
    lcol = jax.lax.broadcasted_iota(jnp.int32, (S, 128), 1)
    rhs = jnp.where(lcol == 0, srow >> 5,
                    jnp.where(lcol == 1, srow & 31,
                              jnp.where(lcol == 2, 1, 0))).astype(jnp.bfloat16)
    mv = jax.lax.dot_general(m.astype(jnp.bfloat16), rhs,
                             (((1,), (0,)), ((), ())),
                             preferred_element_type=jnp.float32)
    aidx = (mv[:, 0:1] * 32.0 + mv[:, 1:2]).astype(jnp.int32)
    cnt = mv[:, 2:3]

    dbits = jax.lax.bitcast_convert_type(d, jnp.int32)
    aidx_ref[...] = aidx.reshape(P // 128, 128)
    dbits_ref[...] = dbits.reshape(P // 128, 128)

    @pl.when(jnp.max(cnt) > 1.5)
    def _():
        sidx = jax.lax.broadcasted_iota(jnp.int32, (P, S), 1)
        aidx_x = jnp.min(jnp.where(m, sidx, jnp.int32(S)),
                         axis=1, keepdims=True)
        aidx_ref[...] = aidx_x.reshape(P // 128, 128)


def _sc_body(aidx_hbm, dbits_hbm, pbd_hbm, pbi_hbm, av, dv, bd, bi):
    C = av.shape[0]
    S = pbd_hbm.shape[1] // _L
    wid = jax.lax.axis_index("c") * 16 + jax.lax.axis_index("s")
    pltpu.sync_copy(aidx_hbm.at[pl.ds(wid * C, C)], av)
    pltpu.sync_copy(dbits_hbm.at[pl.ds(wid * C, C)], dv)

    fmax16 = jnp.full((_L,), _FMAXBITS, jnp.int32)

    def init_step(j, carry):
        bd[pl.ds(j * _L, _L)] = fmax16
        return carry

    jax.lax.fori_loop(0, bd.shape[0] // _L, init_step, 0)

    lane = jax.lax.iota(jnp.int32, _L)
    slot0 = lane * S

    def step(g, carry):
        a = av[pl.ds(g * _L, _L)]
        db = dv[pl.ds(g * _L, _L)]
        slot = slot0 + a
        cur = plsc.load_gather(bd, [slot])
        m = db < cur
        plsc.store_scatter(bd, [slot], db, mask=m)
        gidx = wid * C + g * _L + lane
        plsc.store_scatter(bi, [slot], gidx, mask=m)
        return carry

    jax.lax.fori_loop(0, C // _L, step, 0)

    pltpu.sync_copy(bd, pbd_hbm.at[wid])
    pltpu.sync_copy(bi, pbi_hbm.at[wid])


def _combine_body(pbd_ref, pbi_ref, out_ref):
    pbd = pbd_ref[...]
    pbi = pbi_ref[...]
    accd = jnp.min(pbd, axis=0, keepdims=True)
    eq = pbd == accd
    bi_sel = jnp.min(jnp.where(eq, pbi, _IMAX),
                     axis=0, keepdims=True)
    out_ref[...] = jnp.where(accd == _FMAXBITS, jnp.int32(-1), bi_sel)


def kernel(points, anchors):
    N = points.shape[0]
    S = anchors.shape[1]
    aidx, dbits = pl.pallas_call(
        _assign_body,
        grid=(N // _P,),
        in_specs=[
            pl.BlockSpec((_P, 3), lambda i: (i, 0)),
            pl.BlockSpec((3, S), lambda i: (0, 0)),
        ],
        out_specs=[
            pl.BlockSpec((_P // 128, 128), lambda i: (i, 0)),
            pl.BlockSpec((_P // 128, 128), lambda i: (i, 0)),
        ],
        out_shape=[
            jax.ShapeDtypeStruct((N // 128, 128), jnp.int32),
            jax.ShapeDtypeStruct((N // 128, 128), jnp.int32),
        ],
    )(points, anchors)

    C = N // _W
    mesh = plsc.VectorSubcoreMesh(core_axis_name="c", subcore_axis_name="s")
    pbd, pbi = pl.kernel(
        _sc_body,
        out_type=[
            jax.ShapeDtypeStruct((_W, _L * S), jnp.int32),
            jax.ShapeDtypeStruct((_W, _L * S), jnp.int32),
        ],
        mesh=mesh,
        compiler_params=pltpu.CompilerParams(needs_layout_passes=False),
        scratch_types=[
            pltpu.VMEM((C,), jnp.int32),
            pltpu.VMEM((C,), jnp.int32),
            pltpu.VMEM((_L * S,), jnp.int32),
            pltpu.VMEM((_L * S,), jnp.int32),
        ],
    )(aidx.reshape(N), dbits.reshape(N))

    out = pl.pallas_call(
        _combine_body,
        out_shape=jax.ShapeDtypeStruct((1, S), jnp.int32),
    )(pbd.reshape(_W * _L, S), pbi.reshape(_W * _L, S))
    return out.reshape(S)


# speedup vs baseline: 1.3573x; 1.3573x over previous
import jax
import jax.numpy as jnp
import numpy as np
from jax.experimental import pallas as pl
from jax.experimental.pallas import tpu as pltpu
from jax.experimental.pallas import tpu_sc as plsc

_FMAX = np.float32(np.finfo(np.float32).max)
_FMAXBITS = np.int32(np.float32(_FMAX).view(np.int32))
_IMAX = np.int32(2**31 - 1)
_P = 2048
_W = 32
_L = 16


def _assign_body(pts_ref, anch_ref, aidx_ref, dbits_ref):
    P = pts_ref.shape[0]
    S = anch_ref.shape[1]

    pts = pts_ref[...]
    x = pts[:, 0:1]
    y = pts[:, 1:2]
    z = pts[:, 2:3]
    d = jnp.sqrt((x * x + y * y) + z * z)
    n = jnp.where(d == 0.0, jnp.float32(1.0), d)
    tp = (pts / n).astype(jnp.bfloat16)
    ab = anch_ref[...].astype(jnp.bfloat16)
    cos = jax.lax.dot_general(tp, ab, (((1,), (0,)), ((), ())),
                              preferred_element_type=jnp.float32)

    amax = jnp.max(cos, axis=1, keepdims=True)
    sidx = jax.lax.broadcasted_iota(jnp.int32, (P, S), 1)
    aidx = jnp.min(jnp.where(cos == amax, sidx, jnp.int32(S)),
                   axis=1, keepdims=True)

    dbits = jax.lax.bitcast_convert_type(d, jnp.int32)
    aidx_ref[...] = aidx.reshape(P // 128, 128)
    dbits_ref[...] = dbits.reshape(P // 128, 128)


def _sc_body(aidx_hbm, dbits_hbm, pbd_hbm, pbi_hbm, av, dv, bd, bi):
    C = av.shape[0]
    S = pbd_hbm.shape[1] // _L
    wid = jax.lax.axis_index("c") * 16 + jax.lax.axis_index("s")
    pltpu.sync_copy(aidx_hbm.at[pl.ds(wid * C, C)], av)
    pltpu.sync_copy(dbits_hbm.at[pl.ds(wid * C, C)], dv)

    fmax16 = jnp.full((_L,), _FMAXBITS, jnp.int32)

    def init_step(j, carry):
        bd[pl.ds(j * _L, _L)] = fmax16
        return carry

    jax.lax.fori_loop(0, bd.shape[0] // _L, init_step, 0)

    lane = jax.lax.iota(jnp.int32, _L)
    slot0 = lane * S

    def step(g, carry):
        a = av[pl.ds(g * _L, _L)]
        db = dv[pl.ds(g * _L, _L)]
        slot = slot0 + a
        cur = plsc.load_gather(bd, [slot])
        m = db < cur
        plsc.store_scatter(bd, [slot], db, mask=m)
        gidx = wid * C + g * _L + lane
        plsc.store_scatter(bi, [slot], gidx, mask=m)
        return carry

    jax.lax.fori_loop(0, C // _L, step, 0)

    pltpu.sync_copy(bd, pbd_hbm.at[wid])
    pltpu.sync_copy(bi, pbi_hbm.at[wid])


def _combine_body(pbd_ref, pbi_ref, out_ref):
    pbd = pbd_ref[...]
    pbi = pbi_ref[...]
    accd = jnp.min(pbd, axis=0, keepdims=True)
    eq = pbd == accd
    bi_sel = jnp.min(jnp.where(eq, pbi, _IMAX),
                     axis=0, keepdims=True)
    out_ref[...] = jnp.where(accd == _FMAXBITS, jnp.int32(-1), bi_sel)


def kernel(points, anchors):
    N = points.shape[0]
    S = anchors.shape[1]
    aidx, dbits = pl.pallas_call(
        _assign_body,
        grid=(N // _P,),
        in_specs=[
            pl.BlockSpec((_P, 3), lambda i: (i, 0)),
            pl.BlockSpec((3, S), lambda i: (0, 0)),
        ],
        out_specs=[
            pl.BlockSpec((_P // 128, 128), lambda i: (i, 0)),
            pl.BlockSpec((_P // 128, 128), lambda i: (i, 0)),
        ],
        out_shape=[
            jax.ShapeDtypeStruct((N // 128, 128), jnp.int32),
            jax.ShapeDtypeStruct((N // 128, 128), jnp.int32),
        ],
    )(points, anchors)

    C = N // _W
    mesh = plsc.VectorSubcoreMesh(core_axis_name="c", subcore_axis_name="s")
    pbd, pbi = pl.kernel(
        _sc_body,
        out_type=[
            jax.ShapeDtypeStruct((_W, _L * S), jnp.int32),
            jax.ShapeDtypeStruct((_W, _L * S), jnp.int32),
        ],
        mesh=mesh,
        compiler_params=pltpu.CompilerParams(needs_layout_passes=False),
        scratch_types=[
            pltpu.VMEM((C,), jnp.int32),
            pltpu.VMEM((C,), jnp.int32),
            pltpu.VMEM((_L * S,), jnp.int32),
            pltpu.VMEM((_L * S,), jnp.int32),
        ],
    )(aidx.reshape(N), dbits.reshape(N))

    out = pl.pallas_call(
        _combine_body,
        out_shape=jax.ShapeDtypeStruct((1, S), jnp.int32),
    )(pbd.reshape(_W * _L, S), pbi.reshape(_W * _L, S))
    return out.reshape(S)


# speedup vs baseline: 1.3616x; 1.0031x over previous
import jax
import jax.numpy as jnp
import numpy as np
from jax.experimental import pallas as pl
from jax.experimental.pallas import tpu as pltpu
from jax.experimental.pallas import tpu_sc as plsc

_FMAX = np.float32(np.finfo(np.float32).max)
_FMAXBITS = np.int32(np.float32(_FMAX).view(np.int32))
_IMAX = np.int32(2**31 - 1)
_P = 4096
_W = 32
_L = 16


def _assign_body(pts_ref, anch_ref, aidx_ref, dbits_ref):
    P = pts_ref.shape[0]
    S = anch_ref.shape[1]

    pts = pts_ref[...]
    x = pts[:, 0:1]
    y = pts[:, 1:2]
    z = pts[:, 2:3]
    d = jnp.sqrt((x * x + y * y) + z * z)
    n = jnp.where(d == 0.0, jnp.float32(1.0), d)
    tp = (pts / n).astype(jnp.bfloat16)
    ab = anch_ref[...].astype(jnp.bfloat16)
    cos = jax.lax.dot_general(tp, ab, (((1,), (0,)), ((), ())),
                              preferred_element_type=jnp.float32)

    amax = jnp.max(cos, axis=1, keepdims=True)
    sidx = jax.lax.broadcasted_iota(jnp.int32, (P, S), 1)
    aidx = jnp.min(jnp.where(cos == amax, sidx, jnp.int32(S)),
                   axis=1, keepdims=True)

    dbits = jax.lax.bitcast_convert_type(d, jnp.int32)
    aidx_ref[...] = aidx.reshape(P // 128, 128)
    dbits_ref[...] = dbits.reshape(P // 128, 128)


def _sc_body(aidx_hbm, dbits_hbm, pbd_hbm, pbi_hbm, av, dv, bd, bi):
    C = av.shape[0]
    S = pbd_hbm.shape[1] // _L
    wid = jax.lax.axis_index("c") * 16 + jax.lax.axis_index("s")
    pltpu.sync_copy(aidx_hbm.at[pl.ds(wid * C, C)], av)
    pltpu.sync_copy(dbits_hbm.at[pl.ds(wid * C, C)], dv)

    fmax16 = jnp.full((_L,), _FMAXBITS, jnp.int32)

    def init_step(j, carry):
        bd[pl.ds(j * _L, _L)] = fmax16
        return carry

    jax.lax.fori_loop(0, bd.shape[0] // _L, init_step, 0)

    lane = jax.lax.iota(jnp.int32, _L)
    slot0 = lane * S

    def step(g, carry):
        a = av[pl.ds(g * _L, _L)]
        db = dv[pl.ds(g * _L, _L)]
        slot = slot0 + a
        cur = plsc.load_gather(bd, [slot])
        m = db < cur
        plsc.store_scatter(bd, [slot], db, mask=m)
        gidx = wid * C + g * _L + lane
        plsc.store_scatter(bi, [slot], gidx, mask=m)
        return carry

    jax.lax.fori_loop(0, C // _L, step, 0)

    pltpu.sync_copy(bd, pbd_hbm.at[wid])
    pltpu.sync_copy(bi, pbi_hbm.at[wid])


def _combine_body(pbd_ref, pbi_ref, out_ref):
    pbd = pbd_ref[...]
    pbi = pbi_ref[...]
    accd = jnp.min(pbd, axis=0, keepdims=True)
    eq = pbd == accd
    bi_sel = jnp.min(jnp.where(eq, pbi, _IMAX),
                     axis=0, keepdims=True)
    out_ref[...] = jnp.where(accd == _FMAXBITS, jnp.int32(-1), bi_sel)


def kernel(points, anchors):
    N = points.shape[0]
    S = anchors.shape[1]
    aidx, dbits = pl.pallas_call(
        _assign_body,
        grid=(N // _P,),
        in_specs=[
            pl.BlockSpec((_P, 3), lambda i: (i, 0)),
            pl.BlockSpec((3, S), lambda i: (0, 0)),
        ],
        out_specs=[
            pl.BlockSpec((_P // 128, 128), lambda i: (i, 0)),
            pl.BlockSpec((_P // 128, 128), lambda i: (i, 0)),
        ],
        out_shape=[
            jax.ShapeDtypeStruct((N // 128, 128), jnp.int32),
            jax.ShapeDtypeStruct((N // 128, 128), jnp.int32),
        ],
    )(points, anchors)

    C = N // _W
    mesh = plsc.VectorSubcoreMesh(core_axis_name="c", subcore_axis_name="s")
    pbd, pbi = pl.kernel(
        _sc_body,
        out_type=[
            jax.ShapeDtypeStruct((_W, _L * S), jnp.int32),
            jax.ShapeDtypeStruct((_W, _L * S), jnp.int32),
        ],
        mesh=mesh,
        compiler_params=pltpu.CompilerParams(needs_layout_passes=False),
        scratch_types=[
            pltpu.VMEM((C,), jnp.int32),
            pltpu.VMEM((C,), jnp.int32),
            pltpu.VMEM((_L * S,), jnp.int32),
            pltpu.VMEM((_L * S,), jnp.int32),
        ],
    )(aidx.reshape(N), dbits.reshape(N))

    out = pl.pallas_call(
        _combine_body,
        out_shape=jax.ShapeDtypeStruct((1, S), jnp.int32),
    )(pbd.reshape(_W * _L, S), pbi.reshape(_W * _L, S))
    return out.reshape(S)


# speedup vs baseline: 1.8864x; 1.3855x over previous
import jax
import jax.numpy as jnp
import numpy as np
from jax.experimental import pallas as pl
from jax.experimental.pallas import tpu as pltpu
from jax.experimental.pallas import tpu_sc as plsc

_FMAX = np.float32(np.finfo(np.float32).max)
_FMAXBITS = np.int32(np.float32(_FMAX).view(np.int32))
_IMAX = np.int32(2**31 - 1)
_P = 2048
_W = 32
_L = 16


def _assign_body(xs_ref, ys_ref, zs_ref, anchT_ref, aidx_ref, dbits_ref):
    P = xs_ref.shape[2]
    S = anchT_ref.shape[0]

    x = xs_ref[0]
    y = ys_ref[0]
    z = zs_ref[0]
    d = jnp.sqrt((x * x + y * y) + z * z)
    n = jnp.where(d == 0.0, jnp.float32(1.0), d)
    tpT = (jnp.concatenate([x, y, z], axis=0) / n).astype(jnp.bfloat16)
    abT = anchT_ref[...].astype(jnp.bfloat16)
    cosT = jax.lax.dot_general(abT, tpT, (((1,), (0,)), ((), ())),
                               preferred_element_type=jnp.float32)

    amax = jnp.max(cosT, axis=0, keepdims=True)
    sidx = jax.lax.broadcasted_iota(jnp.int32, (S, P), 0)
    aidx = jnp.min(jnp.where(cosT == amax, sidx, jnp.int32(S)),
                   axis=0, keepdims=True)

    aidx_ref[...] = aidx[None]
    dbits_ref[...] = jax.lax.bitcast_convert_type(d, jnp.int32)[None]


def _sc_body(aidx_hbm, dbits_hbm, pbd_hbm, pbi_hbm, av, dv, bd, bi):
    C = av.shape[0]
    S = pbd_hbm.shape[1] // _L
    wid = jax.lax.axis_index("c") * 16 + jax.lax.axis_index("s")
    pltpu.sync_copy(aidx_hbm.at[pl.ds(wid * C, C)], av)
    pltpu.sync_copy(dbits_hbm.at[pl.ds(wid * C, C)], dv)

    fmax16 = jnp.full((_L,), _FMAXBITS, jnp.int32)

    def init_step(j, carry):
        bd[pl.ds(j * _L, _L)] = fmax16
        return carry

    jax.lax.fori_loop(0, bd.shape[0] // _L, init_step, 0)

    lane = jax.lax.iota(jnp.int32, _L)
    slot0 = lane * S

    def step(g, carry):
        a = av[pl.ds(g * _L, _L)]
        db = dv[pl.ds(g * _L, _L)]
        slot = slot0 + a
        cur = plsc.load_gather(bd, [slot])
        m = db < cur
        plsc.store_scatter(bd, [slot], db, mask=m)
        gidx = wid * C + g * _L + lane
        plsc.store_scatter(bi, [slot], gidx, mask=m)
        return carry

    jax.lax.fori_loop(0, C // _L, step, 0)

    pltpu.sync_copy(bd, pbd_hbm.at[wid])
    pltpu.sync_copy(bi, pbi_hbm.at[wid])


def _combine_body(pbd_ref, pbi_ref, out_ref):
    pbd = pbd_ref[...]
    pbi = pbi_ref[...]
    accd = jnp.min(pbd, axis=0, keepdims=True)
    eq = pbd == accd
    bi_sel = jnp.min(jnp.where(eq, pbi, _IMAX),
                     axis=0, keepdims=True)
    out_ref[...] = jnp.where(accd == _FMAXBITS, jnp.int32(-1), bi_sel)


def kernel(points, anchors):
    N = points.shape[0]
    S = anchors.shape[1]
    nb = N // _P
    xs = points[:, 0].reshape(nb, 1, _P)
    ys = points[:, 1].reshape(nb, 1, _P)
    zs = points[:, 2].reshape(nb, 1, _P)
    coord_spec = pl.BlockSpec((1, 1, _P), lambda i: (i, 0, 0))
    aidx, dbits = pl.pallas_call(
        _assign_body,
        grid=(nb,),
        in_specs=[
            coord_spec,
            coord_spec,
            coord_spec,
            pl.BlockSpec((S, 3), lambda i: (0, 0)),
        ],
        out_specs=[
            pl.BlockSpec((1, 1, _P), lambda i: (i, 0, 0)),
            pl.BlockSpec((1, 1, _P), lambda i: (i, 0, 0)),
        ],
        out_shape=[
            jax.ShapeDtypeStruct((nb, 1, _P), jnp.int32),
            jax.ShapeDtypeStruct((nb, 1, _P), jnp.int32),
        ],
    )(xs, ys, zs, anchors.T)

    C = N // _W
    mesh = plsc.VectorSubcoreMesh(core_axis_name="c", subcore_axis_name="s")
    pbd, pbi = pl.kernel(
        _sc_body,
        out_type=[
            jax.ShapeDtypeStruct((_W, _L * S), jnp.int32),
            jax.ShapeDtypeStruct((_W, _L * S), jnp.int32),
        ],
        mesh=mesh,
        compiler_params=pltpu.CompilerParams(needs_layout_passes=False),
        scratch_types=[
            pltpu.VMEM((C,), jnp.int32),
            pltpu.VMEM((C,), jnp.int32),
            pltpu.VMEM((_L * S,), jnp.int32),
            pltpu.VMEM((_L * S,), jnp.int32),
        ],
    )(aidx.reshape(N), dbits.reshape(N))

    out = pl.pallas_call(
        _combine_body,
        out_shape=jax.ShapeDtypeStruct((1, S), jnp.int32),
    )(pbd.reshape(_W * _L, S), pbi.reshape(_W * _L, S))
    return out.reshape(S)


# speedup vs baseline: 1.9421x; 1.0295x over previous
import jax
import jax.numpy as jnp
import numpy as np
from jax.experimental import pallas as pl
from jax.experimental.pallas import tpu as pltpu
from jax.experimental.pallas import tpu_sc as plsc

_FMAX = np.float32(np.finfo(np.float32).max)
_FMAXBITS = np.int32(np.float32(_FMAX).view(np.int32))
_IMAX = np.int32(2**31 - 1)
_P = 4096
_W = 32
_L = 16


def _assign_body(xs_ref, ys_ref, zs_ref, anchT_ref, aidx_ref, dbits_ref):
    P = xs_ref.shape[2]
    S = anchT_ref.shape[0]

    x = xs_ref[0]
    y = ys_ref[0]
    z = zs_ref[0]
    d = jnp.sqrt((x * x + y * y) + z * z)
    n = jnp.where(d == 0.0, jnp.float32(1.0), d)
    tpT = (jnp.concatenate([x, y, z], axis=0) / n).astype(jnp.bfloat16)
    abT = anchT_ref[...].astype(jnp.bfloat16)
    cosT = jax.lax.dot_general(abT, tpT, (((1,), (0,)), ((), ())),
                               preferred_element_type=jnp.float32)

    amax = jnp.max(cosT, axis=0, keepdims=True)
    sidx = jax.lax.broadcasted_iota(jnp.int32, (S, P), 0)
    aidx = jnp.min(jnp.where(cosT == amax, sidx, jnp.int32(S)),
                   axis=0, keepdims=True)

    aidx_ref[...] = aidx[None]
    dbits_ref[...] = jax.lax.bitcast_convert_type(d, jnp.int32)[None]


def _sc_body(aidx_hbm, dbits_hbm, pbd_hbm, pbi_hbm, av, dv, bd, bi):
    C = av.shape[0]
    S = pbd_hbm.shape[1] // _L
    wid = jax.lax.axis_index("c") * 16 + jax.lax.axis_index("s")
    pltpu.sync_copy(aidx_hbm.at[pl.ds(wid * C, C)], av)
    pltpu.sync_copy(dbits_hbm.at[pl.ds(wid * C, C)], dv)

    fmax16 = jnp.full((_L,), _FMAXBITS, jnp.int32)

    def init_step(j, carry):
        bd[pl.ds(j * _L, _L)] = fmax16
        return carry

    jax.lax.fori_loop(0, bd.shape[0] // _L, init_step, 0)

    lane = jax.lax.iota(jnp.int32, _L)
    slot0 = lane * S

    def step(g, carry):
        a = av[pl.ds(g * _L, _L)]
        db = dv[pl.ds(g * _L, _L)]
        slot = slot0 + a
        cur = plsc.load_gather(bd, [slot])
        m = db < cur
        plsc.store_scatter(bd, [slot], db, mask=m)
        gidx = wid * C + g * _L + lane
        plsc.store_scatter(bi, [slot], gidx, mask=m)
        return carry

    jax.lax.fori_loop(0, C // _L, step, 0)

    pltpu.sync_copy(bd, pbd_hbm.at[wid])
    pltpu.sync_copy(bi, pbi_hbm.at[wid])


def _combine_body(pbd_ref, pbi_ref, out_ref):
    pbd = pbd_ref[...]
    pbi = pbi_ref[...]
    accd = jnp.min(pbd, axis=0, keepdims=True)
    eq = pbd == accd
    bi_sel = jnp.min(jnp.where(eq, pbi, _IMAX),
                     axis=0, keepdims=True)
    out_ref[...] = jnp.where(accd == _FMAXBITS, jnp.int32(-1), bi_sel)


def kernel(points, anchors):
    N = points.shape[0]
    S = anchors.shape[1]
    nb = N // _P
    xs = points[:, 0].reshape(nb, 1, _P)
    ys = points[:, 1].reshape(nb, 1, _P)
    zs = points[:, 2].reshape(nb, 1, _P)
    coord_spec = pl.BlockSpec((1, 1, _P), lambda i: (i, 0, 0))
    aidx, dbits = pl.pallas_call(
        _assign_body,
        grid=(nb,),
        in_specs=[
            coord_spec,
            coord_spec,
            coord_spec,
            pl.BlockSpec((S, 3), lambda i: (0, 0)),
        ],
        out_specs=[
            pl.BlockSpec((1, 1, _P), lambda i: (i, 0, 0)),
            pl.BlockSpec((1, 1, _P), lambda i: (i, 0, 0)),
        ],
        out_shape=[
            jax.ShapeDtypeStruct((nb, 1, _P), jnp.int32),
            jax.ShapeDtypeStruct((nb, 1, _P), jnp.int32),
        ],
    )(xs, ys, zs, anchors.T)

    C = N // _W
    mesh = plsc.VectorSubcoreMesh(core_axis_name="c", subcore_axis_name="s")
    pbd, pbi = pl.kernel(
        _sc_body,
        out_type=[
            jax.ShapeDtypeStruct((_W, _L * S), jnp.int32),
            jax.ShapeDtypeStruct((_W, _L * S), jnp.int32),
        ],
        mesh=mesh,
        compiler_params=pltpu.CompilerParams(needs_layout_passes=False),
        scratch_types=[
            pltpu.VMEM((C,), jnp.int32),
            pltpu.VMEM((C,), jnp.int32),
            pltpu.VMEM((_L * S,), jnp.int32),
            pltpu.VMEM((_L * S,), jnp.int32),
        ],
    )(aidx.reshape(N), dbits.reshape(N))

    out = pl.pallas_call(
        _combine_body,
        out_shape=jax.ShapeDtypeStruct((1, S), jnp.int32),
    )(pbd.reshape(_W * _L, S), pbi.reshape(_W * _L, S))
    return out.reshape(S)
